# baseline (speedup 1.0000x reference)
import jax
import jax.numpy as jnp
from jax.experimental import pallas as pl
from jax.experimental.pallas import tpu as pltpu
from jax.experimental.pallas import tpu_sc as plsc

NUM_COLS = 26
VOCAB = 1000
D_MODEL = 64
BATCH = 4096
STRIDE = 1024
TOTAL = BATCH * NUM_COLS
WINDOW = 128
CGRID = 2
CCOLS = NUM_COLS // CGRID


def _combine_body(tables_ref, col_ref, comb_ref):
    for k in range(CCOLS):
        comb_ref[k * STRIDE : k * STRIDE + VOCAB + 1, :] = (
            tables_ref[k] + col_ref[k]
        )


_combine = pl.pallas_call(
    _combine_body,
    grid=(CGRID,),
    in_specs=[
        pl.BlockSpec((CCOLS, VOCAB + 1, D_MODEL), lambda c: (c, 0, 0)),
        pl.BlockSpec((CCOLS, 1, D_MODEL), lambda c: (c, 0, 0)),
    ],
    out_specs=pl.BlockSpec((CCOLS * STRIDE, D_MODEL), lambda c: (c, 0)),
    out_shape=jax.ShapeDtypeStruct((NUM_COLS * STRIDE, D_MODEL), jnp.float32),
)

def _pad_idx_body(x_cat_ref, idx_ref):
    col_ids = jax.lax.broadcasted_iota(jnp.int32, (BATCH // 8, NUM_COLS), 1)
    idx_ref[:, :NUM_COLS] = x_cat_ref[...] + col_ids * STRIDE


_pad_idx = pl.pallas_call(
    _pad_idx_body,
    grid=(8,),
    in_specs=[pl.BlockSpec((BATCH // 8, NUM_COLS), lambda i: (i, 0))],
    out_specs=pl.BlockSpec((BATCH // 8, 128), lambda i: (i, 0)),
    out_shape=jax.ShapeDtypeStruct((BATCH, 128), jnp.int32),
)

_mesh = plsc.VectorSubcoreMesh(core_axis_name="c", subcore_axis_name="s")

BROWS = 32
WROWS = BROWS * NUM_COLS
NCHUNK = WROWS // 16
GSLICE = 104
LANES = 16


@pl.kernel(
    out_type=jax.ShapeDtypeStruct((TOTAL, D_MODEL), jnp.float32),
    mesh=_mesh,
    compiler_params=pltpu.CompilerParams(
        use_tc_tiling_on_sc=False, needs_layout_passes=False
    ),
    scratch_types=[
        pltpu.VMEM((WROWS,), jnp.int32),
        pltpu.VMEM((WROWS,), jnp.int32),
        pltpu.VMEM((WROWS,), jnp.int32),
        pltpu.SemaphoreType.DMA,
    ],
)
def _sc_gather(x_hbm, table_hbm, out_hbm, rows_v, cols_v, idx_v, sem):
    @pl.loop(0, WROWS, step=LANES)
    def _(j):
        p = jax.lax.iota(jnp.int32, LANES) + j
        rows_v[pl.ds(j, LANES)] = p // NUM_COLS
        cols_v[pl.ds(j, LANES)] = jax.lax.rem(p, NUM_COLS)

    def body(x_vmem, o_vmem):
        @pl.loop(0, WROWS, step=LANES)
        def _(j):
            r = rows_v[pl.ds(j, LANES)]
            c = cols_v[pl.ds(j, LANES)]
            idx_v[pl.ds(j, LANES)] = plsc.load_gather(x_vmem, [r, c])

        copies = [
            pltpu.make_async_copy(
                table_hbm.at[idx_v.at[pl.ds(k * GSLICE, GSLICE)]],
                o_vmem.at[pl.ds(k * GSLICE, GSLICE)],
                sem,
            )
            for k in range(WROWS // GSLICE)
        ]
        for c in copies:
            c.start()
        for c in copies:
            c.wait()

    pltpu.emit_pipeline(
        body,
        grid=(BATCH // BROWS,),
        in_specs=[pl.BlockSpec((BROWS, 128), index_map=lambda i: (i, 0))],
        out_specs=[pl.BlockSpec((WROWS, D_MODEL), index_map=lambda i: (i, 0))],
        core_axis_name=("c", "s"),
        dimension_semantics=(pltpu.PARALLEL,),
    )(x_hbm, out_hbm)


def kernel(x_cat, tables, col_type):
    comb = _combine(tables, col_type.reshape(NUM_COLS, 1, D_MODEL))
    idx128 = _pad_idx(x_cat.astype(jnp.int32))
    flat = _sc_gather(idx128, comb)
    return flat.reshape(BATCH, NUM_COLS, D_MODEL)
